# baseline (speedup 1.0000x reference)
import functools

import jax
import jax.numpy as jnp
from jax import lax
from jax.experimental import pallas as pl
from jax.experimental.pallas import tpu as pltpu
from jax.experimental.pallas import tpu_sc as plsc

VOCAB = 1000
EMBED_DIM = 32
BATCH = 16384
HIST = 50
OROWS = HIST * EMBED_DIM

NUM_CORES = 2
NUM_SUBCORES = 16
NW = NUM_CORES * NUM_SUBCORES
COLS = BATCH // NW
SCOLS = COLS
NGROUP = COLS // 16
NPAIR = HIST // 2
HALF = EMBED_DIM // 2

_mesh = plsc.VectorSubcoreMesh(core_axis_name="c", subcore_axis_name="s")


@functools.partial(
    pl.kernel,
    mesh=_mesh,
    out_type=jax.ShapeDtypeStruct((OROWS, BATCH), jnp.float32),
    compiler_params=pltpu.CompilerParams(needs_layout_passes=False),
    scratch_types=[
        pltpu.VMEM((VOCAB * EMBED_DIM,), jnp.float32),
        pltpu.VMEM((HIST, COLS), jnp.int32),
        pltpu.VMEM((2, EMBED_DIM, SCOLS), jnp.float32),
        pltpu.SemaphoreType.DMA,
        pltpu.SemaphoreType.DMA,
        pltpu.SemaphoreType.DMA,
    ],
)
def _sc_embed(xt_hbm, tablet_hbm, out_hbm, tablet_v, idx_v, stage, w0, w1, tsem):
    wid = lax.axis_index("s") * NUM_CORES + lax.axis_index("c")
    col0 = wid * COLS

    th = pltpu.async_copy(tablet_hbm, tablet_v, tsem)
    ih = pltpu.async_copy(xt_hbm.at[:, pl.ds(col0, COLS)], idx_v, tsem)
    th.wait()
    ih.wait()

    iota16 = lax.iota(jnp.int32, 16)
    dvec_lo = iota16
    dvec_hi = iota16 + 16

    def gather_h(h, bsel):
        @plsc.parallel_loop(0, NGROUP, unroll=2)
        def group_body(g):
            iv = idx_v[h, pl.ds(g * 16, 16)]
            for u in range(16):
                base = iv[u] * EMBED_DIM
                v0 = tablet_v[pl.ds(base, HALF)]
                v1 = tablet_v[pl.ds(base + HALF, HALF)]
                cv = jnp.full((16,), g * 16 + u, jnp.int32)
                plsc.store_scatter(stage.at[bsel], [dvec_lo, cv], v0)
                plsc.store_scatter(stage.at[bsel], [dvec_hi, cv], v1)

    def write_h(h, bsel, sem):
        pltpu.async_copy(
            stage.at[bsel],
            out_hbm.at[pl.ds(h * EMBED_DIM, EMBED_DIM), pl.ds(col0, COLS)],
            sem,
        )

    def drain(sem):
        pltpu.make_async_copy(
            stage.at[0],
            out_hbm.at[pl.ds(0, EMBED_DIM), pl.ds(col0, COLS)],
            sem,
        ).wait()

    def pair_body(p, carry):
        h0 = p * 2

        @pl.when(p > 0)
        def _():
            drain(w0)

        gather_h(h0, 0)
        write_h(h0, 0, w0)

        @pl.when(p > 0)
        def _():
            drain(w1)

        gather_h(h0 + 1, 1)
        write_h(h0 + 1, 1, w1)
        return carry

    lax.fori_loop(0, NPAIR, pair_body, 0)
    drain(w0)
    drain(w1)


def kernel(x, item_id_table):
    out = _sc_embed(x.T, item_id_table.reshape(VOCAB * EMBED_DIM))
    return out.T.reshape(BATCH, HIST, EMBED_DIM)


# speedup vs baseline: 3.4810x; 3.4810x over previous
import functools

import jax
import jax.numpy as jnp
from jax import lax
from jax.experimental import pallas as pl
from jax.experimental.pallas import tpu as pltpu
from jax.experimental.pallas import tpu_sc as plsc

VOCAB = 1000
EMBED_DIM = 32
BATCH = 16384
HIST = 50
OROWS = HIST * EMBED_DIM

NUM_CORES = 2
NUM_SUBCORES = 16
NW = NUM_CORES * NUM_SUBCORES
COLS = BATCH // NW
NGROUP = COLS // 16
NPAIR = HIST // 2

_mesh = plsc.VectorSubcoreMesh(core_axis_name="c", subcore_axis_name="s")


@functools.partial(
    pl.kernel,
    mesh=_mesh,
    out_type=jax.ShapeDtypeStruct((OROWS, BATCH), jnp.float32),
    compiler_params=pltpu.CompilerParams(needs_layout_passes=False),
    scratch_types=[
        pltpu.VMEM((EMBED_DIM * VOCAB,), jnp.float32),
        pltpu.VMEM((HIST, COLS), jnp.int32),
        pltpu.VMEM((2, EMBED_DIM, COLS), jnp.float32),
        pltpu.SemaphoreType.DMA,
        pltpu.SemaphoreType.DMA,
        pltpu.SemaphoreType.DMA,
    ],
)
def _sc_embed(xt_hbm, tablet_hbm, out_hbm, tablet_v, idx_v, stage, w0, w1, tsem):
    wid = lax.axis_index("s") * NUM_CORES + lax.axis_index("c")
    col0 = wid * COLS

    th = pltpu.async_copy(tablet_hbm, tablet_v, tsem)
    ih = pltpu.async_copy(xt_hbm.at[:, pl.ds(col0, COLS)], idx_v, tsem)
    th.wait()
    ih.wait()

    def gather_h(h, bsel):
        @plsc.parallel_loop(0, NGROUP, unroll=8)
        def group_body(g):
            iv = idx_v[h, pl.ds(g * 16, 16)]
            for d in range(EMBED_DIM):
                vals = plsc.load_gather(tablet_v, [iv + d * VOCAB])
                stage[bsel, d, pl.ds(g * 16, 16)] = vals

    def write_h(h, bsel, sem):
        pltpu.async_copy(
            stage.at[bsel],
            out_hbm.at[pl.ds(h * EMBED_DIM, EMBED_DIM), pl.ds(col0, COLS)],
            sem,
        )

    def drain(sem):
        pltpu.make_async_copy(
            stage.at[0],
            out_hbm.at[pl.ds(0, EMBED_DIM), pl.ds(col0, COLS)],
            sem,
        ).wait()

    def pair_body(p, carry):
        h0 = p * 2

        @pl.when(p > 0)
        def _():
            drain(w0)

        gather_h(h0, 0)
        write_h(h0, 0, w0)

        @pl.when(p > 0)
        def _():
            drain(w1)

        gather_h(h0 + 1, 1)
        write_h(h0 + 1, 1, w1)
        return carry

    lax.fori_loop(0, NPAIR, pair_body, 0)
    drain(w0)
    drain(w1)


def kernel(x, item_id_table):
    out = _sc_embed(x.T, item_id_table.T.reshape(EMBED_DIM * VOCAB))
    return out.T.reshape(BATCH, HIST, EMBED_DIM)
